# Initial kernel scaffold; baseline (speedup 1.0000x reference)
import jax
import jax.numpy as jnp
from jax.experimental import pallas as pl


def kernel(predictions, anchor_boxes):
    raise NotImplementedError("write your pallas kernel here")


# speedup vs baseline: 2.7601x; 2.7601x over previous
import numpy as np
import jax
import jax.numpy as jnp
from jax.experimental import pallas as pl
from jax.experimental.pallas import tpu as pltpu

_NUM_CLASSES = 4
_CONF_T = 0.05
_MAX_PER_CLASS = 100
_MAX_DET = 100
_SIGMA = 0.05
_NEG = -3.0e38


def _nms_kernel(pred_ref, anch_ref, outf_ref, outc_ref, *, n_real, npad):
    C = _NUM_CLASSES
    T = _MAX_PER_CLASS

    p = pred_ref[0]
    a = anch_ref[...]

    cxa = a[0:1, :]
    cya = a[1:2, :]
    wa = a[2:3, :]
    ha = a[3:4, :]

    x = p[0:1, :] * wa + cxa
    y = p[1:2, :] * ha + cya
    bw = jnp.exp(p[2:3, :]) * wa
    bh = jnp.exp(p[3:4, :]) * ha
    x1 = x - bw / 2.0
    y1 = y - bh / 2.0
    x2 = x + bw / 2.0
    y2 = y + bh / 2.0
    area = (x2 - x1) * (y2 - y1)

    scores0 = jax.nn.sigmoid(p[4:8, :])

    lane = jax.lax.broadcasted_iota(jnp.int32, (C, npad), 1)
    cid = jax.lax.broadcasted_iota(jnp.int32, (C, npad), 0)
    valid_lane = lane < n_real

    max_score = jnp.max(scores0, axis=0, keepdims=True)
    is_mx = scores0 == max_score
    max_cls = jnp.min(jnp.where(is_mx, cid, C), axis=0, keepdims=True)

    active0_b = (max_score >= _CONF_T) & (max_cls == cid) & valid_lane
    active0 = jnp.where(active0_b, 1.0, 0.0).astype(jnp.float32)

    def body(t, carry):
        scores, active, done, sel_s, sel_v, sx1, sy1, sx2, sy2 = carry
        active_b = active > 0.5
        masked = jnp.where(active_b, scores, -1.0)
        m = jnp.max(masked, axis=1, keepdims=True)
        idx = jnp.min(jnp.where(masked == m, lane, npad), axis=1, keepdims=True)
        ok = jnp.logical_and(done < 0.5, m >= _CONF_T)

        oh = lane == idx
        bx1 = jnp.max(jnp.where(oh, x1, _NEG), axis=1, keepdims=True)
        by1 = jnp.max(jnp.where(oh, y1, _NEG), axis=1, keepdims=True)
        bx2 = jnp.max(jnp.where(oh, x2, _NEG), axis=1, keepdims=True)
        by2 = jnp.max(jnp.where(oh, y2, _NEG), axis=1, keepdims=True)

        ix1 = jnp.maximum(bx1, x1)
        iy1 = jnp.maximum(by1, y1)
        ix2 = jnp.minimum(bx2, x2)
        iy2 = jnp.minimum(by2, y2)
        inter = jnp.maximum(ix2 - ix1, 0.0) * jnp.maximum(iy2 - iy1, 0.0)
        a_sel = (bx2 - bx1) * (by2 - by1)
        union = a_sel + area - inter
        iou = jnp.where(union > 0.0, inter / jnp.maximum(union, 1e-12), 0.0)

        rem = jnp.logical_and(active_b, lane != idx)
        decayed = scores * jnp.exp(-(iou * iou) / _SIGMA)
        new_scores = jnp.where(jnp.logical_and(ok, rem), decayed, scores)
        keep_b = jnp.logical_and(rem, new_scores >= _CONF_T)
        new_active = jnp.where(ok, jnp.where(keep_b, 1.0, 0.0), active)
        new_done = jnp.where(ok, jnp.zeros_like(done), jnp.ones_like(done))

        lt = jax.lax.broadcasted_iota(jnp.int32, (C, 128), 1) == t
        okl = jnp.logical_and(lt, ok)
        sel_s = jnp.where(okl, m, sel_s)
        sel_v = jnp.where(lt, jnp.where(ok, 1.0, 0.0), sel_v)
        sx1 = jnp.where(okl, bx1, sx1)
        sy1 = jnp.where(okl, by1, sy1)
        sx2 = jnp.where(okl, bx2, sx2)
        sy2 = jnp.where(okl, by2, sy2)
        return (new_scores, new_active, new_done, sel_s, sel_v, sx1, sy1, sx2, sy2)

    init = (
        scores0,
        active0,
        jnp.zeros((C, 1), dtype=jnp.float32),
        jnp.zeros((C, 128), dtype=jnp.float32),
        jnp.zeros((C, 128), dtype=jnp.float32),
        jnp.zeros((C, 128), dtype=jnp.float32),
        jnp.zeros((C, 128), dtype=jnp.float32),
        jnp.zeros((C, 128), dtype=jnp.float32),
        jnp.zeros((C, 128), dtype=jnp.float32),
    )
    (_, _, _, sel_s, sel_v, sx1, sy1, sx2, sy2) = jax.lax.fori_loop(
        0, T, body, init
    )

    cid8 = jax.lax.broadcasted_iota(jnp.int32, (C, 128), 0)
    g = cid8 * 128 + jax.lax.broadcasted_iota(jnp.int32, (C, 128), 1)
    g_f = g.astype(jnp.float32)
    nvalid = jnp.sum(sel_v)
    case_b = nvalid > float(_MAX_DET)

    primary = jnp.where(case_b, sel_s, -g_f)
    l128 = jax.lax.broadcasted_iota(jnp.int32, (1, 128), 1)

    def mbody(j, carry):
        R, ox1, oy1, ox2, oy2, osc, ocl = carry
        R_b = R > 0.5
        pm = jnp.where(R_b, primary, _NEG)
        m2 = jnp.max(pm)
        any_rem = m2 > (_NEG * 0.5)
        cand = jnp.logical_and(R_b, pm == m2)
        g_sel = jnp.min(jnp.where(cand, g, 1 << 30))
        oh2 = g == g_sel
        vx1 = jnp.max(jnp.where(oh2, sx1, _NEG))
        vy1 = jnp.max(jnp.where(oh2, sy1, _NEG))
        vx2 = jnp.max(jnp.where(oh2, sx2, _NEG))
        vy2 = jnp.max(jnp.where(oh2, sy2, _NEG))
        vsc = jnp.max(jnp.where(oh2, sel_s, _NEG))
        vcl = jnp.max(jnp.where(oh2, cid8, -1))
        new_R = jnp.where(jnp.logical_and(oh2, any_rem), 0.0, R)
        ohj = jnp.logical_and(l128 == j, any_rem)
        ox1 = jnp.where(ohj, vx1, ox1)
        oy1 = jnp.where(ohj, vy1, oy1)
        ox2 = jnp.where(ohj, vx2, ox2)
        oy2 = jnp.where(ohj, vy2, oy2)
        osc = jnp.where(ohj, vsc, osc)
        ocl = jnp.where(ohj, vcl, ocl)
        return (new_R, ox1, oy1, ox2, oy2, osc, ocl)

    minit = (
        sel_v,
        jnp.zeros((1, 128), dtype=jnp.float32),
        jnp.zeros((1, 128), dtype=jnp.float32),
        jnp.zeros((1, 128), dtype=jnp.float32),
        jnp.zeros((1, 128), dtype=jnp.float32),
        jnp.zeros((1, 128), dtype=jnp.float32),
        jnp.full((1, 128), -1, dtype=jnp.int32),
    )
    (_, ox1, oy1, ox2, oy2, osc, ocl) = jax.lax.fori_loop(0, _MAX_DET, mbody, minit)

    zf = jnp.zeros((3, 128), dtype=jnp.float32)
    outf_ref[0] = jnp.concatenate([ox1, oy1, ox2, oy2, osc, zf], axis=0)
    zi = jnp.zeros((7, 128), dtype=jnp.int32)
    outc_ref[0] = jnp.concatenate([ocl, zi], axis=0)


def kernel(predictions, anchor_boxes):
    B, n, _ = predictions.shape
    npad = ((n + 127) // 128) * 128

    predT = jnp.transpose(predictions, (0, 2, 1))
    predT = jnp.pad(predT, ((0, 0), (0, 0), (0, npad - n)))
    anchT = jnp.pad(anchor_boxes.T, ((0, 0), (0, npad - n)))

    import functools
    kfn = functools.partial(_nms_kernel, n_real=n, npad=npad)
    outf, outc = pl.pallas_call(
        kfn,
        grid=(B,),
        in_specs=[
            pl.BlockSpec((1, 8, npad), lambda b: (b, 0, 0)),
            pl.BlockSpec((4, npad), lambda b: (0, 0)),
        ],
        out_specs=[
            pl.BlockSpec((1, 8, 128), lambda b: (b, 0, 0)),
            pl.BlockSpec((1, 8, 128), lambda b: (b, 0, 0)),
        ],
        out_shape=[
            jax.ShapeDtypeStruct((B, 8, 128), jnp.float32),
            jax.ShapeDtypeStruct((B, 8, 128), jnp.int32),
        ],
        compiler_params=pltpu.CompilerParams(
            dimension_semantics=("arbitrary",),
        ),
    )(predT, anchT)

    M = _MAX_DET
    boxes = jnp.stack(
        [outf[:, 0, :M], outf[:, 1, :M], outf[:, 2, :M], outf[:, 3, :M]], axis=-1
    )
    scores = outf[:, 4, :M]
    classes = outc[:, 0, :M]
    valid = jnp.sum((classes > -1).astype(jnp.int32), axis=1)
    idt = jax.dtypes.canonicalize_dtype(np.int64)
    return (
        valid.astype(jnp.int32),
        boxes.astype(jnp.float32),
        scores.astype(jnp.float32),
        classes.astype(idt),
    )
